# baseline (speedup 1.0000x reference)
import functools

import jax
import jax.numpy as jnp
from jax.experimental import pallas as pl
from jax.experimental.pallas import tpu as pltpu


def _round_up(n, m):
    return ((n + m - 1) // m) * m


def _gather_body(bw_ref,
                 kw_ref,
                 bc_ref,
                 kc_ref,
                 wv_ref,
                 lr_ref,
                 out_ref,
                 *, tm, ws, d):
    i = pl.program_id(0)
    base = i * tm
    sub = jax.lax.broadcasted_iota(jnp.int32, (8, 1), 0)

    def group(g, carry):
        b = base + g * 8
        accw = acce = None
        for u in range(8):
            w = wv_ref[pl.ds(pl.multiple_of(bw_ref[b + u], 8), 8), :]
            e = lr_ref[pl.ds(pl.multiple_of(bc_ref[b + u], 8), 8), :]
            wr = pltpu.roll(w, kw_ref[b + u], 0)
            er = pltpu.roll(e, kc_ref[b + u], 0)
            if u == 0:
                accw, acce = wr, er
            else:
                m = sub == u
                accw = jnp.where(m, wr, accw)
                acce = jnp.where(m, er, acce)
        t0 = pl.multiple_of(g * 8, 8)
        out_ref[pl.ds(t0, 8), 0:ws] = accw
        out_ref[pl.ds(t0, 8), ws:d] = acce
        return carry

    jax.lax.fori_loop(0, tm // 8, group, 0)


@jax.jit
def kernel(x, ldist, rdist, Wv, pf1, pf2):
    B, S = x.shape
    WL, WS = Wv.shape
    FL, FS = pf1.shape
    FL2, FS2 = pf2.shape
    LR = FS + FS2
    D = WS + LR
    N = B * S

    xi = jnp.clip(x.reshape(N).astype(jnp.int32), 0, WL - 1)
    li = jnp.clip(ldist.reshape(N).astype(jnp.int32), 0, FL - 1)
    ri = jnp.clip(rdist.reshape(N).astype(jnp.int32), 0, FL2 - 1)
    ci = li * FL2 + ri

    tm = min(1024, _round_up(N, 8))
    n_pad = _round_up(N, tm)
    pad = n_pad - N
    if pad:
        zero = jnp.zeros((pad,), jnp.int32)
        xi = jnp.concatenate([xi, zero])
        ci = jnp.concatenate([ci, zero])

    tpos = jax.lax.iota(jnp.int32, n_pad) & 7
    bw = xi & ~7
    kw = (tpos - xi) & 7
    bc = ci & ~7
    kc = (tpos - ci) & 7

    lr_tab = jnp.concatenate(
        [jnp.broadcast_to(pf1[:, None, :], (FL, FL2, FS)),
         jnp.broadcast_to(pf2[None, :, :], (FL, FL2, FS2))],
        axis=-1).reshape(FL * FL2, LR)

    vmem_bytes = (WL * WS * 4 + FL * FL2 * LR * 4 + 2 * tm * D * 4
                  + (1 << 20))
    out = pl.pallas_call(
        functools.partial(_gather_body, tm=tm, ws=WS, d=D),
        out_shape=jax.ShapeDtypeStruct((n_pad, D), jnp.float32),
        grid_spec=pltpu.PrefetchScalarGridSpec(
            num_scalar_prefetch=4,
            grid=(n_pad // tm,),
            in_specs=[
                pl.BlockSpec((WL, WS), lambda i, *_: (0, 0)),
                pl.BlockSpec((FL * FL2, LR), lambda i, *_: (0, 0)),
            ],
            out_specs=pl.BlockSpec((tm, D), lambda i, *_: (i, 0)),
        ),
        compiler_params=pltpu.CompilerParams(
            dimension_semantics=("arbitrary",),
            vmem_limit_bytes=min(vmem_bytes, 60 * 1024 * 1024),
        ),
    )(bw, kw, bc, kc, Wv, lr_tab)

    return out[:N].reshape(B, S, D)[:, None, :, :]


# speedup vs baseline: 1.1682x; 1.1682x over previous
import functools

import jax
import jax.numpy as jnp
from jax.experimental import pallas as pl
from jax.experimental.pallas import tpu as pltpu


def _round_up(n, m):
    return ((n + m - 1) // m) * m


_UG = 2


def _gather_body(pw_ref,
                 pc_ref,
                 wv_ref,
                 lr_ref,
                 out_ref,
                 *, tm, ws, d):
    i = pl.program_id(0)
    base = i * tm
    sub = jax.lax.broadcasted_iota(jnp.int32, (8, 1), 0)

    def one_group(b, t0):
        accw = acce = None
        for u in range(8):
            pw = pw_ref[b + u]
            pc = pc_ref[b + u]
            w = wv_ref[pl.ds(pl.multiple_of(pw >> 3, 8), 8), :]
            e = lr_ref[pl.ds(pl.multiple_of(pc >> 3, 8), 8), :]
            wr = pltpu.roll(w, pw & 7, 0)
            er = pltpu.roll(e, pc & 7, 0)
            if u == 0:
                accw, acce = wr, er
            else:
                m = sub == u
                accw = jnp.where(m, wr, accw)
                acce = jnp.where(m, er, acce)
        out_ref[pl.ds(t0, 8), 0:ws] = accw
        out_ref[pl.ds(t0, 8), ws:d] = acce

    def block(g, carry):
        for j in range(_UG):
            t0 = pl.multiple_of(g * (8 * _UG) + 8 * j, 8)
            one_group(base + g * (8 * _UG) + 8 * j, t0)
        return carry

    jax.lax.fori_loop(0, tm // (8 * _UG), block, 0)


@jax.jit
def kernel(x, ldist, rdist, Wv, pf1, pf2):
    B, S = x.shape
    WL, WS = Wv.shape
    FL, FS = pf1.shape
    FL2, FS2 = pf2.shape
    LR = FS + FS2
    D = WS + LR
    N = B * S

    xi = jnp.clip(x.reshape(N).astype(jnp.int32), 0, WL - 1)
    li = jnp.clip(ldist.reshape(N).astype(jnp.int32), 0, FL - 1)
    ri = jnp.clip(rdist.reshape(N).astype(jnp.int32), 0, FL2 - 1)
    ci = li * FL2 + ri

    tm = min(1024, _round_up(N, 8))
    n_pad = _round_up(N, tm)
    pad = n_pad - N
    if pad:
        zero = jnp.zeros((pad,), jnp.int32)
        xi = jnp.concatenate([xi, zero])
        ci = jnp.concatenate([ci, zero])

    tpos = jax.lax.iota(jnp.int32, n_pad) & 7
    pw = ((xi & ~7) << 3) | ((tpos - xi) & 7)
    pc = ((ci & ~7) << 3) | ((tpos - ci) & 7)

    lr_tab = jnp.concatenate(
        [jnp.broadcast_to(pf1[:, None, :], (FL, FL2, FS)),
         jnp.broadcast_to(pf2[None, :, :], (FL, FL2, FS2))],
        axis=-1).reshape(FL * FL2, LR)

    vmem_bytes = (WL * WS * 4 + FL * FL2 * LR * 4 + 2 * tm * D * 4
                  + (1 << 20))
    out = pl.pallas_call(
        functools.partial(_gather_body, tm=tm, ws=WS, d=D),
        out_shape=jax.ShapeDtypeStruct((n_pad, D), jnp.float32),
        grid_spec=pltpu.PrefetchScalarGridSpec(
            num_scalar_prefetch=2,
            grid=(n_pad // tm,),
            in_specs=[
                pl.BlockSpec((WL, WS), lambda i, *_: (0, 0)),
                pl.BlockSpec((FL * FL2, LR), lambda i, *_: (0, 0)),
            ],
            out_specs=pl.BlockSpec((tm, D), lambda i, *_: (i, 0)),
        ),
        compiler_params=pltpu.CompilerParams(
            dimension_semantics=("arbitrary",),
            vmem_limit_bytes=min(vmem_bytes, 60 * 1024 * 1024),
        ),
    )(pw, pc, Wv, lr_tab)

    return out[:N].reshape(B, S, D)[:, None, :, :]


# speedup vs baseline: 1.2199x; 1.0442x over previous
import functools

import jax
import jax.numpy as jnp
from jax.experimental import pallas as pl
from jax.experimental.pallas import tpu as pltpu


def _round_up(n, m):
    return ((n + m - 1) // m) * m


_UG = 4


def _gather_body(pw_ref,
                 pc_ref,
                 wv_ref,
                 lr_ref,
                 out_ref,
                 *, tm, ws, d):
    i = pl.program_id(0)
    base = i * tm
    sub = jax.lax.broadcasted_iota(jnp.int32, (8, 1), 0)

    def one_group(b, t0):
        accw = acce = None
        for u in range(8):
            pw = pw_ref[b + u]
            pc = pc_ref[b + u]
            w = wv_ref[pl.ds(pl.multiple_of(pw >> 3, 8), 8), :]
            e = lr_ref[pl.ds(pl.multiple_of(pc >> 3, 8), 8), :]
            wr = pltpu.roll(w, pw & 7, 0)
            er = pltpu.roll(e, pc & 7, 0)
            if u == 0:
                accw, acce = wr, er
            else:
                m = sub == u
                accw = jnp.where(m, wr, accw)
                acce = jnp.where(m, er, acce)
        out_ref[pl.ds(t0, 8), 0:ws] = accw
        out_ref[pl.ds(t0, 8), ws:d] = acce

    def block(g, carry):
        for j in range(_UG):
            t0 = pl.multiple_of(g * (8 * _UG) + 8 * j, 8)
            one_group(base + g * (8 * _UG) + 8 * j, t0)
        return carry

    jax.lax.fori_loop(0, tm // (8 * _UG), block, 0)


@jax.jit
def kernel(x, ldist, rdist, Wv, pf1, pf2):
    B, S = x.shape
    WL, WS = Wv.shape
    FL, FS = pf1.shape
    FL2, FS2 = pf2.shape
    LR = FS + FS2
    D = WS + LR
    N = B * S

    xi = jnp.clip(x.reshape(N).astype(jnp.int32), 0, WL - 1)
    li = jnp.clip(ldist.reshape(N).astype(jnp.int32), 0, FL - 1)
    ri = jnp.clip(rdist.reshape(N).astype(jnp.int32), 0, FL2 - 1)
    ci = li * FL2 + ri

    tm = min(1024, _round_up(N, 8))
    n_pad = _round_up(N, tm)
    pad = n_pad - N
    if pad:
        zero = jnp.zeros((pad,), jnp.int32)
        xi = jnp.concatenate([xi, zero])
        ci = jnp.concatenate([ci, zero])

    tpos = jax.lax.iota(jnp.int32, n_pad) & 7
    pw = ((xi & ~7) << 3) | ((tpos - xi) & 7)
    pc = ((ci & ~7) << 3) | ((tpos - ci) & 7)

    lr_tab = jnp.concatenate(
        [jnp.broadcast_to(pf1[:, None, :], (FL, FL2, FS)),
         jnp.broadcast_to(pf2[None, :, :], (FL, FL2, FS2))],
        axis=-1).reshape(FL * FL2, LR)

    vmem_bytes = (WL * WS * 4 + FL * FL2 * LR * 4 + 2 * tm * D * 4
                  + (1 << 20))
    out = pl.pallas_call(
        functools.partial(_gather_body, tm=tm, ws=WS, d=D),
        out_shape=jax.ShapeDtypeStruct((n_pad, D), jnp.float32),
        grid_spec=pltpu.PrefetchScalarGridSpec(
            num_scalar_prefetch=2,
            grid=(n_pad // tm,),
            in_specs=[
                pl.BlockSpec((WL, WS), lambda i, *_: (0, 0)),
                pl.BlockSpec((FL * FL2, LR), lambda i, *_: (0, 0)),
            ],
            out_specs=pl.BlockSpec((tm, D), lambda i, *_: (i, 0)),
        ),
        compiler_params=pltpu.CompilerParams(
            dimension_semantics=("arbitrary",),
            vmem_limit_bytes=min(vmem_bytes, 60 * 1024 * 1024),
        ),
    )(pw, pc, Wv, lr_tab)

    return out[:N].reshape(B, S, D)[:, None, :, :]


# speedup vs baseline: 1.6145x; 1.3235x over previous
import functools

import jax
import jax.numpy as jnp
from jax.experimental import pallas as pl
from jax.experimental.pallas import tpu as pltpu


def _round_up(n, m):
    return ((n + m - 1) // m) * m


_UG = 4


def _gather_body(pw_ref,
                 lrix_ref,
                 wv_ref,
                 tab_ref,
                 out_ref,
                 *, tm, ws, d):
    i = pl.program_id(0)
    base = i * tm

    v = tab_ref.shape[0]
    lr = lrix_ref[...]
    iota = jax.lax.broadcasted_iota(jnp.int32, (tm, v), 1)
    onehot = ((lr[:, 0:1] == iota) | (lr[:, 1:2] == iota)).astype(jnp.float32)
    out_ref[:, ws:d] = jnp.dot(onehot, tab_ref[...],
                               preferred_element_type=jnp.float32)

    sub = jax.lax.broadcasted_iota(jnp.int32, (8, 1), 0)

    def one_group(b, t0):
        accw = None
        for u in range(8):
            pw = pw_ref[b + u]
            w = wv_ref[pl.ds(pl.multiple_of(pw >> 3, 8), 8), :]
            wr = pltpu.roll(w, pw & 7, 0)
            accw = wr if u == 0 else jnp.where(sub == u, wr, accw)
        out_ref[pl.ds(t0, 8), 0:ws] = accw

    def block(g, carry):
        for j in range(_UG):
            t0 = pl.multiple_of(g * (8 * _UG) + 8 * j, 8)
            one_group(base + g * (8 * _UG) + 8 * j, t0)
        return carry

    jax.lax.fori_loop(0, tm // (8 * _UG), block, 0)


@jax.jit
def kernel(x, ldist, rdist, Wv, pf1, pf2):
    B, S = x.shape
    WL, WS = Wv.shape
    FL, FS = pf1.shape
    FL2, FS2 = pf2.shape
    LR = FS + FS2
    D = WS + LR
    N = B * S

    xi = jnp.clip(x.reshape(N).astype(jnp.int32), 0, WL - 1)
    li = jnp.clip(ldist.reshape(N).astype(jnp.int32), 0, FL - 1)
    ri = jnp.clip(rdist.reshape(N).astype(jnp.int32), 0, FL2 - 1)

    tm = min(1024, _round_up(N, 8))
    n_pad = _round_up(N, tm)
    pad = n_pad - N
    if pad:
        zero = jnp.zeros((pad,), jnp.int32)
        xi = jnp.concatenate([xi, zero])
        li = jnp.concatenate([li, zero])
        ri = jnp.concatenate([ri, zero])

    tpos = jax.lax.iota(jnp.int32, n_pad) & 7
    pw = ((xi & ~7) << 3) | ((tpos - xi) & 7)
    lrix = jnp.stack([li, ri + FL], axis=-1)

    tab = jnp.zeros((FL + FL2, LR), jnp.float32)
    tab = tab.at[:FL, :FS].set(pf1.astype(jnp.float32))
    tab = tab.at[FL:, FS:].set(pf2.astype(jnp.float32))

    vmem_bytes = (WL * WS * 4 + (FL + FL2) * LR * 4 + 2 * tm * D * 4
                  + 2 * tm * 128 * 4 + tm * (FL + FL2) * 4 + (1 << 20))
    out = pl.pallas_call(
        functools.partial(_gather_body, tm=tm, ws=WS, d=D),
        out_shape=jax.ShapeDtypeStruct((n_pad, D), jnp.float32),
        grid_spec=pltpu.PrefetchScalarGridSpec(
            num_scalar_prefetch=1,
            grid=(n_pad // tm,),
            in_specs=[
                pl.BlockSpec((tm, 2), lambda i, *_: (i, 0)),
                pl.BlockSpec((WL, WS), lambda i, *_: (0, 0)),
                pl.BlockSpec((FL + FL2, LR), lambda i, *_: (0, 0)),
            ],
            out_specs=pl.BlockSpec((tm, D), lambda i, *_: (i, 0)),
        ),
        compiler_params=pltpu.CompilerParams(
            dimension_semantics=("arbitrary",),
            vmem_limit_bytes=min(vmem_bytes, 60 * 1024 * 1024),
        ),
    )(pw, lrix, Wv, tab)

    return out[:N].reshape(B, S, D)[:, None, :, :]


# speedup vs baseline: 1.7126x; 1.0608x over previous
import functools

import jax
import jax.numpy as jnp
from jax.experimental import pallas as pl
from jax.experimental.pallas import tpu as pltpu


def _round_up(n, m):
    return ((n + m - 1) // m) * m


_UG = 4


def _gather_body(pw_ref,
                 oh_ref,
                 wv_ref,
                 tab_ref,
                 out_ref,
                 *, tm, ws, d):
    i = pl.program_id(0)
    base = i * tm

    out_ref[:, ws:d] = jnp.dot(oh_ref[...], tab_ref[...],
                               preferred_element_type=jnp.float32)

    sub = jax.lax.broadcasted_iota(jnp.int32, (8, 1), 0)

    def one_group(b, t0):
        accw = None
        for u in range(8):
            pw = pw_ref[b + u]
            w = wv_ref[pl.ds(pl.multiple_of(pw >> 3, 8), 8), :]
            wr = pltpu.roll(w, pw & 7, 0)
            accw = wr if u == 0 else jnp.where(sub == u, wr, accw)
        out_ref[pl.ds(t0, 8), 0:ws] = accw

    def block(g, carry):
        for j in range(_UG):
            t0 = pl.multiple_of(g * (8 * _UG) + 8 * j, 8)
            one_group(base + g * (8 * _UG) + 8 * j, t0)
        return carry

    jax.lax.fori_loop(0, tm // (8 * _UG), block, 0)


@jax.jit
def kernel(x, ldist, rdist, Wv, pf1, pf2):
    B, S = x.shape
    WL, WS = Wv.shape
    FL, FS = pf1.shape
    FL2, FS2 = pf2.shape
    LR = FS + FS2
    D = WS + LR
    N = B * S

    xi = jnp.clip(x.reshape(N).astype(jnp.int32), 0, WL - 1)
    li = jnp.clip(ldist.reshape(N).astype(jnp.int32), 0, FL - 1)
    ri = jnp.clip(rdist.reshape(N).astype(jnp.int32), 0, FL2 - 1)

    tm = min(1024, _round_up(N, 8))
    n_pad = _round_up(N, tm)
    pad = n_pad - N
    if pad:
        zero = jnp.zeros((pad,), jnp.int32)
        xi = jnp.concatenate([xi, zero])
        li = jnp.concatenate([li, zero])
        ri = jnp.concatenate([ri, zero])

    tpos = jax.lax.iota(jnp.int32, n_pad) & 7
    pw = ((xi & ~7) << 3) | ((tpos - xi) & 7)
    vv = jax.lax.iota(jnp.int32, FL + FL2)
    oh = ((li[:, None] == vv[None, :])
          | ((ri + FL)[:, None] == vv[None, :])).astype(jnp.float32)

    tab = jnp.zeros((FL + FL2, LR), jnp.float32)
    tab = tab.at[:FL, :FS].set(pf1.astype(jnp.float32))
    tab = tab.at[FL:, FS:].set(pf2.astype(jnp.float32))

    vmem_bytes = (WL * WS * 4 + (FL + FL2) * LR * 4 + 2 * tm * D * 4
                  + 2 * tm * 128 * 4 + tm * (FL + FL2) * 4 + (1 << 20))
    out = pl.pallas_call(
        functools.partial(_gather_body, tm=tm, ws=WS, d=D),
        out_shape=jax.ShapeDtypeStruct((n_pad, D), jnp.float32),
        grid_spec=pltpu.PrefetchScalarGridSpec(
            num_scalar_prefetch=1,
            grid=(n_pad // tm,),
            in_specs=[
                pl.BlockSpec((tm, FL + FL2), lambda i, *_: (i, 0)),
                pl.BlockSpec((WL, WS), lambda i, *_: (0, 0)),
                pl.BlockSpec((FL + FL2, LR), lambda i, *_: (0, 0)),
            ],
            out_specs=pl.BlockSpec((tm, D), lambda i, *_: (i, 0)),
        ),
        compiler_params=pltpu.CompilerParams(
            dimension_semantics=("arbitrary",),
            vmem_limit_bytes=min(vmem_bytes, 60 * 1024 * 1024),
        ),
    )(pw, oh, Wv, tab)

    return out[:N].reshape(B, S, D)[:, None, :, :]


# speedup vs baseline: 1.7531x; 1.0236x over previous
import functools

import jax
import jax.numpy as jnp
from jax.experimental import pallas as pl
from jax.experimental.pallas import tpu as pltpu


def _round_up(n, m):
    return ((n + m - 1) // m) * m


_UG = 8


def _gather_body(pw_ref,
                 oh_ref,
                 wv_ref,
                 tab_ref,
                 out_ref,
                 *, tm, ws, d):
    i = pl.program_id(0)
    base = i * tm

    out_ref[:, ws:d] = jnp.dot(oh_ref[...], tab_ref[...],
                               preferred_element_type=jnp.float32)

    sub = jax.lax.broadcasted_iota(jnp.int32, (8, 1), 0)

    def one_group(b, t0):
        accw = None
        for u in range(8):
            pw = pw_ref[b + u]
            w = wv_ref[pl.ds(pl.multiple_of(pw >> 3, 8), 8), :]
            wr = pltpu.roll(w, pw & 7, 0)
            accw = wr if u == 0 else jnp.where(sub == u, wr, accw)
        out_ref[pl.ds(t0, 8), 0:ws] = accw

    def block(g, carry):
        for j in range(_UG):
            t0 = pl.multiple_of(g * (8 * _UG) + 8 * j, 8)
            one_group(base + g * (8 * _UG) + 8 * j, t0)
        return carry

    jax.lax.fori_loop(0, tm // (8 * _UG), block, 0)


@jax.jit
def kernel(x, ldist, rdist, Wv, pf1, pf2):
    B, S = x.shape
    WL, WS = Wv.shape
    FL, FS = pf1.shape
    FL2, FS2 = pf2.shape
    LR = FS + FS2
    D = WS + LR
    N = B * S

    xi = jnp.clip(x.reshape(N).astype(jnp.int32), 0, WL - 1)
    li = jnp.clip(ldist.reshape(N).astype(jnp.int32), 0, FL - 1)
    ri = jnp.clip(rdist.reshape(N).astype(jnp.int32), 0, FL2 - 1)

    tm = min(1024, _round_up(N, 8))
    n_pad = _round_up(N, tm)
    pad = n_pad - N
    if pad:
        zero = jnp.zeros((pad,), jnp.int32)
        xi = jnp.concatenate([xi, zero])
        li = jnp.concatenate([li, zero])
        ri = jnp.concatenate([ri, zero])

    tpos = jax.lax.iota(jnp.int32, n_pad) & 7
    pw = ((xi & ~7) << 3) | ((tpos - xi) & 7)
    vv = jax.lax.iota(jnp.int32, FL + FL2)
    oh = ((li[:, None] == vv[None, :])
          | ((ri + FL)[:, None] == vv[None, :])).astype(jnp.float32)

    tab = jnp.zeros((FL + FL2, LR), jnp.float32)
    tab = tab.at[:FL, :FS].set(pf1.astype(jnp.float32))
    tab = tab.at[FL:, FS:].set(pf2.astype(jnp.float32))

    vmem_bytes = (WL * WS * 4 + (FL + FL2) * LR * 4 + 2 * tm * D * 4
                  + 2 * tm * 128 * 4 + tm * (FL + FL2) * 4 + (1 << 20))
    out = pl.pallas_call(
        functools.partial(_gather_body, tm=tm, ws=WS, d=D),
        out_shape=jax.ShapeDtypeStruct((n_pad, D), jnp.float32),
        grid_spec=pltpu.PrefetchScalarGridSpec(
            num_scalar_prefetch=1,
            grid=(n_pad // tm,),
            in_specs=[
                pl.BlockSpec((tm, FL + FL2), lambda i, *_: (i, 0)),
                pl.BlockSpec((WL, WS), lambda i, *_: (0, 0)),
                pl.BlockSpec((FL + FL2, LR), lambda i, *_: (0, 0)),
            ],
            out_specs=pl.BlockSpec((tm, D), lambda i, *_: (i, 0)),
        ),
        compiler_params=pltpu.CompilerParams(
            dimension_semantics=("arbitrary",),
            vmem_limit_bytes=min(vmem_bytes, 60 * 1024 * 1024),
        ),
    )(pw, oh, Wv, tab)

    return out[:N].reshape(B, S, D)[:, None, :, :]


# speedup vs baseline: 1.7880x; 1.0199x over previous
import functools

import jax
import jax.numpy as jnp
from jax.experimental import pallas as pl
from jax.experimental.pallas import tpu as pltpu


def _round_up(n, m):
    return ((n + m - 1) // m) * m


_UG = 8


def _gather_body(pw_ref,
                 oh_ref,
                 wv_ref,
                 tab_ref,
                 out_ref,
                 *, tm, ws, d):
    i = pl.program_id(0)
    base = i * tm

    out_ref[:, ws:d] = jnp.dot(oh_ref[...], tab_ref[...],
                               preferred_element_type=jnp.float32)

    sub = jax.lax.broadcasted_iota(jnp.int32, (8, 1), 0)

    def one_group(b, t0):
        accw = None
        for u in range(8):
            pw = pw_ref[b + u]
            w = wv_ref[pl.ds(pl.multiple_of(pw >> 3, 8), 8), :]
            wr = pltpu.roll(w, pw & 7, 0)
            accw = wr if u == 0 else jnp.where(sub == u, wr, accw)
        out_ref[pl.ds(t0, 8), 0:ws] = accw

    def block(g, carry):
        for j in range(_UG):
            t0 = pl.multiple_of(g * (8 * _UG) + 8 * j, 8)
            one_group(base + g * (8 * _UG) + 8 * j, t0)
        return carry

    jax.lax.fori_loop(0, tm // (8 * _UG), block, 0)


@jax.jit
def kernel(x, ldist, rdist, Wv, pf1, pf2):
    B, S = x.shape
    WL, WS = Wv.shape
    FL, FS = pf1.shape
    FL2, FS2 = pf2.shape
    LR = FS + FS2
    D = WS + LR
    N = B * S

    xi = jnp.clip(x.reshape(N).astype(jnp.int32), 0, WL - 1)
    li = jnp.clip(ldist.reshape(N).astype(jnp.int32), 0, FL - 1)
    ri = jnp.clip(rdist.reshape(N).astype(jnp.int32), 0, FL2 - 1)

    tm = min(2048, _round_up(N, 8))
    n_pad = _round_up(N, tm)
    pad = n_pad - N
    if pad:
        zero = jnp.zeros((pad,), jnp.int32)
        xi = jnp.concatenate([xi, zero])
        li = jnp.concatenate([li, zero])
        ri = jnp.concatenate([ri, zero])

    tpos = jax.lax.iota(jnp.int32, n_pad) & 7
    pw = ((xi & ~7) << 3) | ((tpos - xi) & 7)
    vv = jax.lax.iota(jnp.int32, FL + FL2)
    oh = ((li[:, None] == vv[None, :])
          | ((ri + FL)[:, None] == vv[None, :])).astype(jnp.float32)

    tab = jnp.zeros((FL + FL2, LR), jnp.float32)
    tab = tab.at[:FL, :FS].set(pf1.astype(jnp.float32))
    tab = tab.at[FL:, FS:].set(pf2.astype(jnp.float32))

    vmem_bytes = (WL * WS * 4 + (FL + FL2) * LR * 4 + 2 * tm * D * 4
                  + 2 * tm * 128 * 4 + tm * (FL + FL2) * 4 + (1 << 20))
    out = pl.pallas_call(
        functools.partial(_gather_body, tm=tm, ws=WS, d=D),
        out_shape=jax.ShapeDtypeStruct((n_pad, D), jnp.float32),
        grid_spec=pltpu.PrefetchScalarGridSpec(
            num_scalar_prefetch=1,
            grid=(n_pad // tm,),
            in_specs=[
                pl.BlockSpec((tm, FL + FL2), lambda i, *_: (i, 0)),
                pl.BlockSpec((WL, WS), lambda i, *_: (0, 0)),
                pl.BlockSpec((FL + FL2, LR), lambda i, *_: (0, 0)),
            ],
            out_specs=pl.BlockSpec((tm, D), lambda i, *_: (i, 0)),
        ),
        compiler_params=pltpu.CompilerParams(
            dimension_semantics=("arbitrary",),
            vmem_limit_bytes=min(vmem_bytes, 60 * 1024 * 1024),
        ),
    )(pw, oh, Wv, tab)

    return out[:N].reshape(B, S, D)[:, None, :, :]
